# baseline (speedup 1.0000x reference)
import functools

import jax
import jax.numpy as jnp
from jax import lax
from jax.experimental import pallas as pl
from jax.experimental.pallas import tpu as pltpu
from jax.experimental.pallas import tpu_sc as plsc

N = 10000
E = 320000
D = 128

NC = 2
NS = 16
NW = NC * NS
EW = E // NW
K = 80
NCHUNK = EW // K
NPAD = 10240
RPT = NPAD // NS
ZR = 128

_mesh = plsc.VectorSubcoreMesh(core_axis_name="c", subcore_axis_name="s")


def _deg_body(dst1_hbm, ones_hbm, zeros1_hbm, out_hbm, didx_v, ones_v, acc_sh,
              sem, lsem):
    c = lax.axis_index("c")
    s = lax.axis_index("s")
    wid = c * NS + s
    pltpu.sync_copy(zeros1_hbm, acc_sh.at[pl.ds(s * RPT, RPT)])
    pltpu.sync_copy(ones_hbm, ones_v)

    GRP = 25

    def ldgroup(g, carry):
        def fire(k, carry2):
            base = pl.multiple_of(wid * EW + (g * GRP + k) * K, 8)
            pltpu.async_copy(dst1_hbm.at[pl.ds(base, K)],
                             didx_v.at[g * GRP + k], lsem)
            return carry2

        lax.fori_loop(0, GRP, fire, 0)

        def drain(k, carry2):
            pltpu.make_async_copy(dst1_hbm.at[pl.ds(wid * EW, K)],
                                  didx_v.at[0], lsem).wait()
            return carry2

        lax.fori_loop(0, GRP, drain, 0)
        return carry

    lax.fori_loop(0, NCHUNK // GRP, ldgroup, 0)
    plsc.subcore_barrier()


    def group(g, carry):
        def fire(k, carry2):
            pltpu.async_copy(ones_v, acc_sh.at[didx_v.at[g * GRP + k]],
                             sem, add=True)
            return carry2

        lax.fori_loop(0, GRP, fire, 0)

        def drain(k, carry2):
            pltpu.make_async_copy(ones_v, acc_sh.at[didx_v.at[g * GRP]],
                                  sem).wait()
            return carry2

        lax.fori_loop(0, GRP, drain, 0)
        return carry

    lax.fori_loop(0, NCHUNK // GRP, group, 0)
    plsc.subcore_barrier()
    pltpu.sync_copy(acc_sh.at[pl.ds(s * RPT, RPT)],
                    out_hbm.at[c, pl.ds(s * RPT, RPT)])


_deg_call = pl.kernel(
    _deg_body,
    out_type=jax.ShapeDtypeStruct((NC, NPAD), jnp.float32),
    mesh=_mesh,
    scratch_types=[
        pltpu.VMEM((NCHUNK, K), jnp.int32),
        pltpu.VMEM((K,), jnp.float32),
        pltpu.VMEM_SHARED((NPAD,), jnp.float32),
        pltpu.SemaphoreType.DMA,
        pltpu.SemaphoreType.DMA,
    ],
)


def _prop_body(t_hbm, src1_hbm, dst1_hbm, zeros_hbm, out_hbm,
               sidx_v, didx_a, didx_b, rows_a, rows_b, acc_sh,
               gsa, gsb, dsa, dsb, ssa, ssb):
    c = lax.axis_index("c")
    s = lax.axis_index("s")
    wid = c * NS + s
    GRP = 25

    def ldgroup(g, carry):
        def fire(k, carry2):
            base = pl.multiple_of(wid * EW + (g * GRP + k) * K, 8)
            pltpu.async_copy(src1_hbm.at[pl.ds(base, K)],
                             sidx_v.at[g * GRP + k], ssa)
            return carry2

        lax.fori_loop(0, GRP, fire, 0)

        def drain(k, carry2):
            pltpu.make_async_copy(src1_hbm.at[pl.ds(wid * EW, K)],
                                  sidx_v.at[0], ssa).wait()
            return carry2

        lax.fori_loop(0, GRP, drain, 0)
        return carry

    lax.fori_loop(0, NCHUNK // GRP, ldgroup, 0)
    for j in range(RPT // ZR):
        pltpu.sync_copy(zeros_hbm, acc_sh.at[pl.ds(s * RPT + j * ZR, ZR)])
    plsc.subcore_barrier()

    pltpu.async_copy(dst1_hbm.at[pl.ds(wid * EW, K)], didx_a, dsa)
    pltpu.async_copy(t_hbm.at[sidx_v.at[0]], rows_a, gsa)

    def body(i, carry):
        k0 = 2 * i
        base = pl.multiple_of(wid * EW + k0 * K, 8)
        pltpu.async_copy(dst1_hbm.at[pl.ds(base + K, K)], didx_b, dsb)
        pltpu.async_copy(t_hbm.at[sidx_v.at[k0 + 1]], rows_b, gsb)
        pltpu.make_async_copy(dst1_hbm.at[pl.ds(base, K)], didx_a, dsa).wait()
        pltpu.make_async_copy(t_hbm.at[sidx_v.at[k0]], rows_a, gsa).wait()
        pltpu.sync_copy(rows_a, acc_sh.at[didx_a], add=True)
        pltpu.async_copy(dst1_hbm.at[pl.ds(base + 2 * K, K)], didx_a, dsa)
        pltpu.async_copy(t_hbm.at[sidx_v.at[k0 + 2]], rows_a, gsa)
        pltpu.make_async_copy(dst1_hbm.at[pl.ds(base + K, K)], didx_b,
                              dsb).wait()
        pltpu.make_async_copy(t_hbm.at[sidx_v.at[k0 + 1]], rows_b,
                              gsb).wait()
        pltpu.sync_copy(rows_b, acc_sh.at[didx_b], add=True)
        return carry

    lax.fori_loop(0, (NCHUNK - 1) // 2, body, 0)
    pltpu.make_async_copy(
        dst1_hbm.at[pl.ds(wid * EW + (NCHUNK - 1) * K, K)], didx_a,
        dsa).wait()
    pltpu.make_async_copy(t_hbm.at[sidx_v.at[NCHUNK - 1]], rows_a, gsa).wait()
    pltpu.sync_copy(rows_a, acc_sh.at[didx_a], add=True)
    plsc.subcore_barrier()
    pltpu.sync_copy(acc_sh.at[pl.ds(s * RPT, RPT)],
                    out_hbm.at[c, pl.ds(s * RPT, RPT)])


_prop_call = pl.kernel(
    _prop_body,
    out_type=jax.ShapeDtypeStruct((NC, NPAD, D), jnp.float32),
    mesh=_mesh,
    scratch_types=[
        pltpu.VMEM((NCHUNK, K), jnp.int32),
        pltpu.VMEM((K,), jnp.int32),
        pltpu.VMEM((K,), jnp.int32),
        pltpu.VMEM((K, D), jnp.float32),
        pltpu.VMEM((K, D), jnp.float32),
        pltpu.VMEM_SHARED((NPAD, D), jnp.float32),
        pltpu.SemaphoreType.DMA,
        pltpu.SemaphoreType.DMA,
        pltpu.SemaphoreType.DMA,
        pltpu.SemaphoreType.DMA,
        pltpu.SemaphoreType.DMA,
        pltpu.SemaphoreType.DMA,
    ],
)


R = 2000
GRID = N // R

_mm = functools.partial(jnp.dot, preferred_element_type=jnp.float32)


def _b1_body(dp_ref, x_ref, dis_ref, y0_ref):
    deg = dp_ref[0] + dp_ref[1] + 1.0
    dis = lax.rsqrt(deg)
    dis_ref[...] = dis
    y0_ref[...] = x_ref[...] * dis


def _b2_body(dis_ref, y0_ref, p0_ref, w1_ref, b1_ref, w2_ref, b2_ref,
             w4_ref, b4_ref, h1_ref, y2_ref, y3a_ref):
    dis = dis_ref[...]
    z = dis * (p0_ref[0] + p0_ref[1] + y0_ref[...])
    h1_ref[...] = jax.nn.relu(_mm(z, w1_ref[...]) + b1_ref[...])
    y2_ref[...] = dis * jax.nn.relu(_mm(z, w2_ref[...]) + b2_ref[...])
    y3a_ref[...] = dis * jax.nn.relu(_mm(z, w4_ref[...]) + b4_ref[...])


def _b3_body(dis_ref, y2_ref, p2_ref, y3a_ref, p3_ref, w3_ref, b3_ref,
             w5_ref, b5_ref, h2_ref, y3b_ref):
    dis = dis_ref[...]
    u2 = dis * (p2_ref[0] + p2_ref[1] + y2_ref[...])
    h2_ref[...] = jax.nn.relu(_mm(u2, w3_ref[...]) + b3_ref[...])
    u3 = dis * (p3_ref[0] + p3_ref[1] + y3a_ref[...])
    h3b = jax.nn.relu(_mm(u3, w5_ref[...]) + b5_ref[...])
    y3b_ref[...] = dis * h3b


def _b4_body(dis_ref, y3b_ref, p4_ref, h1_ref, h2_ref, w6_ref, b6_ref,
             wa_ref, ba_ref, wf_ref, bf_ref, out_ref):
    dis = dis_ref[...]
    u4 = dis * (p4_ref[0] + p4_ref[1] + y3b_ref[...])
    h3 = jax.nn.relu(_mm(u4, w6_ref[...]) + b6_ref[...])
    h1 = h1_ref[...]
    h2 = h2_ref[...]
    wa = wa_ref[...]
    ba = ba_ref[...]
    e1 = _mm(h1, wa) + ba
    e2 = _mm(h2, wa) + ba
    e3 = _mm(h3, wa) + ba
    m = jnp.maximum(jnp.maximum(e1, e2), e3)
    a1 = jnp.exp(e1 - m)
    a2 = jnp.exp(e2 - m)
    a3 = jnp.exp(e3 - m)
    tot = a1 + a2 + a3
    h_meta = (a1 * h1 + a2 * h2 + a3 * h3) / tot
    out_ref[...] = _mm(h_meta + h1, wf_ref[...]) + bf_ref[...]


def _col_spec(r):
    return pl.BlockSpec((r, 1), lambda i: (i, 0))


def _row_spec():
    return pl.BlockSpec((R, D), lambda i: (i, 0))


def _p_spec():
    return pl.BlockSpec((NC, R, D), lambda i: (0, i, 0))


def _w_spec(shape):
    return pl.BlockSpec(shape, lambda i: tuple(0 for _ in shape))


def _mk_call(body, in_specs, out_specs, out_shapes):
    return pl.pallas_call(
        body,
        grid=(GRID,),
        in_specs=in_specs,
        out_specs=out_specs,
        out_shape=out_shapes,
    )


_b1_call = _mk_call(
    _b1_body,
    [pl.BlockSpec((NC, R, 1), lambda i: (0, i, 0)), _row_spec()],
    [_col_spec(R), _row_spec()],
    [jax.ShapeDtypeStruct((N, 1), jnp.float32),
     jax.ShapeDtypeStruct((N, D), jnp.float32)],
)

_b2_call = _mk_call(
    _b2_body,
    [_col_spec(R), _row_spec(), _p_spec(),
     _w_spec((D, D)), _w_spec((1, D)),
     _w_spec((D, D)), _w_spec((1, D)),
     _w_spec((D, D)), _w_spec((1, D))],
    [_row_spec(), _row_spec(), _row_spec()],
    [jax.ShapeDtypeStruct((N, D), jnp.float32)] * 3,
)

_b3_call = _mk_call(
    _b3_body,
    [_col_spec(R), _row_spec(), _p_spec(), _row_spec(), _p_spec(),
     _w_spec((D, D)), _w_spec((1, D)),
     _w_spec((D, D)), _w_spec((1, D))],
    [_row_spec(), _row_spec()],
    [jax.ShapeDtypeStruct((N, D), jnp.float32)] * 2,
)

_b4_call = _mk_call(
    _b4_body,
    [_col_spec(R), _row_spec(), _p_spec(), _row_spec(), _row_spec(),
     _w_spec((D, D)), _w_spec((1, D)),
     _w_spec((D, 1)), _w_spec((1, 1)),
     _w_spec((D, D)), _w_spec((1, D))],
    [_row_spec()],
    [jax.ShapeDtypeStruct((N, D), jnp.float32)],
)


def kernel(x, edge_index, W1, b1, W2, b2, W3, b3, W4, b4, W5, b5, W6, b6,
           Wa, ba, Wf, bf):
    src1 = edge_index[0]
    dst1 = edge_index[1]
    ones_k = jnp.ones((K,), jnp.float32)
    zeros1 = jnp.zeros((RPT,), jnp.float32)
    zeros2 = jnp.zeros((ZR, D), jnp.float32)

    dp = _deg_call(dst1, ones_k, zeros1)
    dp3 = dp.reshape(NC, NPAD, 1)
    dis, y0 = _b1_call(dp3, x)

    p0 = _prop_call(y0, src1, dst1, zeros2)
    h1, y2, y3a = _b2_call(dis, y0, p0, W1, b1.reshape(1, D), W2,
                           b2.reshape(1, D), W4, b4.reshape(1, D))

    p2 = _prop_call(y2, src1, dst1, zeros2)
    p3 = _prop_call(y3a, src1, dst1, zeros2)
    h2, y3b = _b3_call(dis, y2, p2, y3a, p3, W3, b3.reshape(1, D),
                       W5, b5.reshape(1, D))

    p4 = _prop_call(y3b, src1, dst1, zeros2)
    out, = _b4_call(dis, y3b, p4, h1, h2, W6, b6.reshape(1, D),
                    Wa, ba.reshape(1, 1), Wf, bf.reshape(1, D))
    return out


# speedup vs baseline: 1.0321x; 1.0321x over previous
import functools

import jax
import jax.numpy as jnp
from jax import lax
from jax.experimental import pallas as pl
from jax.experimental.pallas import tpu as pltpu
from jax.experimental.pallas import tpu_sc as plsc

N = 10000
E = 320000
D = 128

NC = 2
NS = 16
NW = NC * NS
EW = E // NW
K = 80
NCHUNK = EW // K
NPAD = 10240
RPT = NPAD // NS
ZR = 128

_mesh = plsc.VectorSubcoreMesh(core_axis_name="c", subcore_axis_name="s")


def _deg_body(dst1_hbm, ones_hbm, zeros1_hbm, out_hbm, didx_v, ones_v, acc_sh,
              sem, lsem):
    c = lax.axis_index("c")
    s = lax.axis_index("s")
    wid = c * NS + s
    pltpu.sync_copy(zeros1_hbm, acc_sh.at[pl.ds(s * RPT, RPT)])
    pltpu.sync_copy(ones_hbm, ones_v)

    GRP = 25

    def ldgroup(g, carry):
        def fire(k, carry2):
            base = pl.multiple_of(wid * EW + (g * GRP + k) * K, 8)
            pltpu.async_copy(dst1_hbm.at[pl.ds(base, K)],
                             didx_v.at[g * GRP + k], lsem)
            return carry2

        lax.fori_loop(0, GRP, fire, 0)

        def drain(k, carry2):
            pltpu.make_async_copy(dst1_hbm.at[pl.ds(wid * EW, K)],
                                  didx_v.at[0], lsem).wait()
            return carry2

        lax.fori_loop(0, GRP, drain, 0)
        return carry

    lax.fori_loop(0, NCHUNK // GRP, ldgroup, 0)
    plsc.subcore_barrier()


    def group(g, carry):
        def fire(k, carry2):
            pltpu.async_copy(ones_v, acc_sh.at[didx_v.at[g * GRP + k]],
                             sem, add=True)
            return carry2

        lax.fori_loop(0, GRP, fire, 0)

        def drain(k, carry2):
            pltpu.make_async_copy(ones_v, acc_sh.at[didx_v.at[g * GRP]],
                                  sem).wait()
            return carry2

        lax.fori_loop(0, GRP, drain, 0)
        return carry

    lax.fori_loop(0, NCHUNK // GRP, group, 0)
    plsc.subcore_barrier()
    pltpu.sync_copy(acc_sh.at[pl.ds(s * RPT, RPT)],
                    out_hbm.at[c, pl.ds(s * RPT, RPT)])


_deg_call = pl.kernel(
    _deg_body,
    out_type=jax.ShapeDtypeStruct((NC, NPAD), jnp.float32),
    mesh=_mesh,
    scratch_types=[
        pltpu.VMEM((NCHUNK, K), jnp.int32),
        pltpu.VMEM((K,), jnp.float32),
        pltpu.VMEM_SHARED((NPAD,), jnp.float32),
        pltpu.SemaphoreType.DMA,
        pltpu.SemaphoreType.DMA,
    ],
)


def _prop_body(t_hbm, si3_hbm, dst1_hbm, zeros_hbm, out_hbm,
               sidx_v, didx_a, didx_b, rows_a, rows_b, acc_sh,
               gsa, gsb, dsa, dsb, ssa, ssb):
    c = lax.axis_index("c")
    s = lax.axis_index("s")
    wid = c * NS + s
    for j in range(RPT // ZR):
        pltpu.sync_copy(zeros_hbm, acc_sh.at[pl.ds(s * RPT + j * ZR, ZR)])
    pltpu.sync_copy(si3_hbm.at[pl.ds(wid, 1)], sidx_v)
    plsc.subcore_barrier()

    pltpu.async_copy(dst1_hbm.at[pl.ds(wid * EW, K)], didx_a, dsa)
    pltpu.async_copy(t_hbm.at[sidx_v.at[0, 0]], rows_a, gsa)

    def body(i, carry):
        k0 = 2 * i
        base = pl.multiple_of(wid * EW + k0 * K, 8)
        pltpu.async_copy(dst1_hbm.at[pl.ds(base + K, K)], didx_b, dsb)
        pltpu.async_copy(t_hbm.at[sidx_v.at[0, k0 + 1]], rows_b, gsb)
        pltpu.make_async_copy(dst1_hbm.at[pl.ds(base, K)], didx_a, dsa).wait()
        pltpu.make_async_copy(t_hbm.at[sidx_v.at[0, k0]], rows_a, gsa).wait()
        pltpu.sync_copy(rows_a, acc_sh.at[didx_a], add=True)
        pltpu.async_copy(dst1_hbm.at[pl.ds(base + 2 * K, K)], didx_a, dsa)
        pltpu.async_copy(t_hbm.at[sidx_v.at[0, k0 + 2]], rows_a, gsa)
        pltpu.make_async_copy(dst1_hbm.at[pl.ds(base + K, K)], didx_b,
                              dsb).wait()
        pltpu.make_async_copy(t_hbm.at[sidx_v.at[0, k0 + 1]], rows_b,
                              gsb).wait()
        pltpu.sync_copy(rows_b, acc_sh.at[didx_b], add=True)
        return carry

    lax.fori_loop(0, (NCHUNK - 1) // 2, body, 0)
    pltpu.make_async_copy(
        dst1_hbm.at[pl.ds(wid * EW + (NCHUNK - 1) * K, K)], didx_a,
        dsa).wait()
    pltpu.make_async_copy(t_hbm.at[sidx_v.at[0, NCHUNK - 1]], rows_a, gsa).wait()
    pltpu.sync_copy(rows_a, acc_sh.at[didx_a], add=True)
    plsc.subcore_barrier()
    pltpu.sync_copy(acc_sh.at[pl.ds(s * RPT, RPT)],
                    out_hbm.at[c, pl.ds(s * RPT, RPT)])


_prop_call = pl.kernel(
    _prop_body,
    out_type=jax.ShapeDtypeStruct((NC, NPAD, D), jnp.float32),
    mesh=_mesh,
    scratch_types=[
        pltpu.VMEM((1, NCHUNK, K), jnp.int32),
        pltpu.VMEM((K,), jnp.int32),
        pltpu.VMEM((K,), jnp.int32),
        pltpu.VMEM((K, D), jnp.float32),
        pltpu.VMEM((K, D), jnp.float32),
        pltpu.VMEM_SHARED((NPAD, D), jnp.float32),
        pltpu.SemaphoreType.DMA,
        pltpu.SemaphoreType.DMA,
        pltpu.SemaphoreType.DMA,
        pltpu.SemaphoreType.DMA,
        pltpu.SemaphoreType.DMA,
        pltpu.SemaphoreType.DMA,
    ],
)


R = 2000
GRID = N // R

_mm = functools.partial(jnp.dot, preferred_element_type=jnp.float32)


def _b1_body(dp_ref, x_ref, dis_ref, y0_ref):
    deg = dp_ref[0:1, :] + dp_ref[1:2, :] + 1.0
    disc = jnp.transpose(lax.rsqrt(deg), (1, 0))[:N]
    dis_ref[...] = disc
    y0_ref[...] = x_ref[...] * disc


def _b2_body(dis_ref, y0_ref, p0_ref, w1_ref, b1_ref, w2_ref, b2_ref,
             w4_ref, b4_ref, h1_ref, y2_ref, y3a_ref):
    dis = dis_ref[...]
    z = dis * (p0_ref[0] + p0_ref[1] + y0_ref[...])
    h1_ref[...] = jax.nn.relu(_mm(z, w1_ref[...]) + b1_ref[...])
    y2_ref[...] = dis * jax.nn.relu(_mm(z, w2_ref[...]) + b2_ref[...])
    y3a_ref[...] = dis * jax.nn.relu(_mm(z, w4_ref[...]) + b4_ref[...])


def _b3_body(dis_ref, y2_ref, p2_ref, y3a_ref, p3_ref, w3_ref, b3_ref,
             w5_ref, b5_ref, h2_ref, y3b_ref):
    dis = dis_ref[...]
    u2 = dis * (p2_ref[0] + p2_ref[1] + y2_ref[...])
    h2_ref[...] = jax.nn.relu(_mm(u2, w3_ref[...]) + b3_ref[...])
    u3 = dis * (p3_ref[0] + p3_ref[1] + y3a_ref[...])
    h3b = jax.nn.relu(_mm(u3, w5_ref[...]) + b5_ref[...])
    y3b_ref[...] = dis * h3b


def _b4_body(dis_ref, y3b_ref, p4_ref, h1_ref, h2_ref, w6_ref, b6_ref,
             wa_ref, ba_ref, wf_ref, bf_ref, out_ref):
    dis = dis_ref[...]
    u4 = dis * (p4_ref[0] + p4_ref[1] + y3b_ref[...])
    h3 = jax.nn.relu(_mm(u4, w6_ref[...]) + b6_ref[...])
    h1 = h1_ref[...]
    h2 = h2_ref[...]
    wa = wa_ref[...]
    ba = ba_ref[...]
    e1 = _mm(h1, wa) + ba
    e2 = _mm(h2, wa) + ba
    e3 = _mm(h3, wa) + ba
    m = jnp.maximum(jnp.maximum(e1, e2), e3)
    a1 = jnp.exp(e1 - m)
    a2 = jnp.exp(e2 - m)
    a3 = jnp.exp(e3 - m)
    tot = a1 + a2 + a3
    h_meta = (a1 * h1 + a2 * h2 + a3 * h3) / tot
    out_ref[...] = _mm(h_meta + h1, wf_ref[...]) + bf_ref[...]


def _col_spec(r):
    return pl.BlockSpec((r, 1), lambda i: (i, 0))


def _row_spec():
    return pl.BlockSpec((R, D), lambda i: (i, 0))


def _p_spec():
    return pl.BlockSpec((NC, R, D), lambda i: (0, i, 0))


def _w_spec(shape):
    return pl.BlockSpec(shape, lambda i: tuple(0 for _ in shape))


def _mk_call(body, in_specs, out_specs, out_shapes):
    return pl.pallas_call(
        body,
        grid=(GRID,),
        in_specs=in_specs,
        out_specs=out_specs,
        out_shape=out_shapes,
    )


_b1_call = pl.pallas_call(
    _b1_body,
    grid=(1,),
    in_specs=[pl.BlockSpec((NC, NPAD), lambda i: (0, 0)),
              pl.BlockSpec((N, D), lambda i: (0, 0))],
    out_specs=[pl.BlockSpec((N, 1), lambda i: (0, 0)),
               pl.BlockSpec((N, D), lambda i: (0, 0))],
    out_shape=[jax.ShapeDtypeStruct((N, 1), jnp.float32),
               jax.ShapeDtypeStruct((N, D), jnp.float32)],
)

_b2_call = _mk_call(
    _b2_body,
    [_col_spec(R), _row_spec(), _p_spec(),
     _w_spec((D, D)), _w_spec((1, D)),
     _w_spec((D, D)), _w_spec((1, D)),
     _w_spec((D, D)), _w_spec((1, D))],
    [_row_spec(), _row_spec(), _row_spec()],
    [jax.ShapeDtypeStruct((N, D), jnp.float32)] * 3,
)

_b3_call = _mk_call(
    _b3_body,
    [_col_spec(R), _row_spec(), _p_spec(), _row_spec(), _p_spec(),
     _w_spec((D, D)), _w_spec((1, D)),
     _w_spec((D, D)), _w_spec((1, D))],
    [_row_spec(), _row_spec()],
    [jax.ShapeDtypeStruct((N, D), jnp.float32)] * 2,
)

_b4_call = _mk_call(
    _b4_body,
    [_col_spec(R), _row_spec(), _p_spec(), _row_spec(), _row_spec(),
     _w_spec((D, D)), _w_spec((1, D)),
     _w_spec((D, 1)), _w_spec((1, 1)),
     _w_spec((D, D)), _w_spec((1, D))],
    [_row_spec()],
    [jax.ShapeDtypeStruct((N, D), jnp.float32)],
)


def kernel(x, edge_index, W1, b1, W2, b2, W3, b3, W4, b4, W5, b5, W6, b6,
           Wa, ba, Wf, bf):
    src3 = edge_index[0].reshape(NW, NCHUNK, K)
    dst1 = edge_index[1]
    ones_k = jnp.ones((K,), jnp.float32)
    zeros1 = jnp.zeros((RPT,), jnp.float32)
    zeros2 = jnp.zeros((ZR, D), jnp.float32)

    dp = _deg_call(dst1, ones_k, zeros1)
    dis, y0 = _b1_call(dp, x)

    p0 = _prop_call(y0, src3, dst1, zeros2)
    h1, y2, y3a = _b2_call(dis, y0, p0, W1, b1.reshape(1, D), W2,
                           b2.reshape(1, D), W4, b4.reshape(1, D))

    p2 = _prop_call(y2, src3, dst1, zeros2)
    p3 = _prop_call(y3a, src3, dst1, zeros2)
    h2, y3b = _b3_call(dis, y2, p2, y3a, p3, W3, b3.reshape(1, D),
                       W5, b5.reshape(1, D))

    p4 = _prop_call(y3b, src3, dst1, zeros2)
    out, = _b4_call(dis, y3b, p4, h1, h2, W6, b6.reshape(1, D),
                    Wa, ba.reshape(1, 1), Wf, bf.reshape(1, D))
    return out
